# baseline (speedup 1.0000x reference)
import functools

import numpy as np
import jax
import jax.numpy as jnp
from jax import lax
from jax.experimental import pallas as pl
from jax.experimental.pallas import tpu as pltpu
from jax.experimental.pallas import tpu_sc as plsc

H = 14
W_ = 14
P = H * W_
PR = 200
WIN_HALF = 3
KK = 9
C = 384
OUT = 384
EPS = 1e-6
NEG = -1e30
NCAND = 64
BPG = 16


def _build_cidx():
    t = np.full((P, NCAND), -1, dtype=np.int32)
    for ki in range(H):
        for kj in range(W_):
            idx = [i * W_ + j
                   for i in range(H) for j in range(W_)
                   if abs(i - ki) <= WIN_HALF and abs(j - kj) <= WIN_HALF]
            t[ki * W_ + kj, :len(idx)] = np.array(idx, dtype=np.int32)
    return t


_CIDX = _build_cidx()


def _sims_body(x_ref, lo_ref, hi_ref):
    for i in range(BPG):
        X = x_ref[i]
        n = jnp.sqrt(jnp.sum(X * X, axis=0))
        S = lax.dot_general(X, X, (((0,), (0,)), ((), ())),
                            preferred_element_type=jnp.float32,
                            precision=lax.Precision.HIGHEST)
        sim = S / jnp.maximum(n[:, None] * n[None, :], EPS)
        lo_ref[pl.ds(i * PR, P)] = sim[:, :128]
        hi_ref[pl.ds(i * PR, P)] = jnp.concatenate(
            [sim[:, 128:], jnp.full((P, 256 - P), NEG, jnp.float32)], axis=1)


def _merge_desc(ak, av, bk, bv):
    brk = lax.rev(bk, (0,))
    brv = lax.rev(bv, (0,))
    m = ak >= brk
    hk = jnp.where(m, ak, brk)
    hv = jnp.where(m, av, brv)
    return plsc.sort_key_val(hk, hv, descending=True)


def _sc_topk_body(lo_hbm, hi_hbm, cidx_hbm, out_hbm,
                  lo_v, hi_v, cidx_v, out_v):
    wid = lax.axis_index("s") * 2 + lax.axis_index("c")
    pltpu.sync_copy(lo_hbm.at[pl.ds(wid * PR, PR)], lo_v)
    pltpu.sync_copy(hi_hbm.at[pl.ds(wid * PR, PR)], hi_v)
    pltpu.sync_copy(cidx_hbm, cidx_v)
    lane = lax.broadcasted_iota(jnp.int32, (16,), 0)

    def body(p, carry):
        psplat = lane * 0 + p
        parts = []
        for k in range(4):
            ci = cidx_v[pl.ds(p * NCAND + 16 * k, 16)]
            valid = ci >= 0
            safe = jnp.where(valid, ci, 0)
            in_lo = safe < 128
            g_lo = plsc.load_gather(lo_v, [psplat, jnp.where(in_lo, safe, 0)])
            g_hi = plsc.load_gather(hi_v, [psplat,
                                           jnp.where(in_lo, 0, safe - 128)])
            g = jnp.where(in_lo, g_lo, g_hi)
            key = jnp.where(valid, g, NEG)
            parts.append(plsc.sort_key_val(key, ci, descending=True))
        k01, v01 = _merge_desc(*parts[0], *parts[1])
        k23, v23 = _merge_desc(*parts[2], *parts[3])
        _, topv = _merge_desc(k01, v01, k23, v23)
        idx9 = jnp.where(lane < KK, topv, jnp.int32(2 ** 30))
        sidx, _ = plsc.sort_key_val(idx9, idx9)
        out_v[pl.ds(p * 128, 16)] = sidx
        return carry

    lax.fori_loop(0, P, body, 0)
    pltpu.sync_copy(out_v, out_hbm.at[pl.ds(wid * PR * 128, PR * 128)])


def _sc_topk(lo, hi, cidx):
    mesh = plsc.VectorSubcoreMesh(core_axis_name="c", subcore_axis_name="s")
    f = pl.kernel(
        _sc_topk_body,
        compiler_params=pltpu.CompilerParams(needs_layout_passes=False),
        out_type=jax.ShapeDtypeStruct((32 * PR * 128,), jnp.int32),
        mesh=mesh,
        scratch_types=[
            pltpu.VMEM((PR, 128), jnp.float32),
            pltpu.VMEM((PR, 128), jnp.float32),
            pltpu.VMEM((P * NCAND,), jnp.int32),
            pltpu.VMEM((PR * 128,), jnp.int32),
        ],
    )
    return f(lo, hi, cidx)


def _conv_body(x_ref, idx_ref, w_ref, o_ref):
    q_i = lax.broadcasted_iota(jnp.int32, (P, P), 1)
    for i in range(BPG):
        X = x_ref[i]
        acc = jnp.zeros((OUT, P), jnp.float32)
        for m in range(KK):
            im = idx_ref[i, :P, m][:, None]
            pm = (q_i == im).astype(jnp.float32)
            gm = lax.dot_general(pm, X, (((1,), (1,)), ((), ())),
                                 preferred_element_type=jnp.float32)
            acc = acc + lax.dot_general(
                w_ref[m], gm, (((0,), (1,)), ((), ())),
                preferred_element_type=jnp.float32)
        o_ref[i] = acc


def kernel(x, W):
    Bn = x.shape[0]
    x_flat = x.reshape(Bn, C, P)
    Wr = jnp.transpose(W.reshape(OUT, C, KK), (2, 1, 0))
    cidx = jnp.asarray(_CIDX.reshape(-1))

    lo, hi = pl.pallas_call(
        _sims_body,
        grid=(Bn // BPG,),
        in_specs=[pl.BlockSpec((BPG, C, P), lambda b: (b, 0, 0))],
        out_specs=[
            pl.BlockSpec((BPG * PR, 128), lambda b: (b, 0)),
            pl.BlockSpec((BPG * PR, 128), lambda b: (b, 0)),
        ],
        out_shape=[
            jax.ShapeDtypeStruct((Bn * PR, 128), jnp.float32),
            jax.ShapeDtypeStruct((Bn * PR, 128), jnp.float32),
        ],
    )(x_flat)

    idx = _sc_topk(lo, hi, cidx).reshape(Bn, PR, 128)

    out = pl.pallas_call(
        _conv_body,
        grid=(Bn // BPG,),
        in_specs=[
            pl.BlockSpec((BPG, C, P), lambda b: (b, 0, 0)),
            pl.BlockSpec((BPG, PR, 128), lambda b: (b, 0, 0)),
            pl.BlockSpec((KK, C, OUT), lambda b: (0, 0, 0)),
        ],
        out_specs=pl.BlockSpec((BPG, OUT, P), lambda b: (b, 0, 0)),
        out_shape=jax.ShapeDtypeStruct((Bn, OUT, P), jnp.float32),
    )(x_flat, idx, Wr)
    return out.reshape(Bn, OUT, H, W_)


# speedup vs baseline: 1.0096x; 1.0096x over previous
import functools

import numpy as np
import jax
import jax.numpy as jnp
from jax import lax
from jax.experimental import pallas as pl
from jax.experimental.pallas import tpu as pltpu
from jax.experimental.pallas import tpu_sc as plsc

H = 14
W_ = 14
P = H * W_
PR = 200
WIN_HALF = 3
KK = 9
C = 384
OUT = 384
EPS = 1e-6
NEG = -1e30
NCAND = 64
BPG = 8


def _build_cidx():
    t = np.full((P, NCAND), -1, dtype=np.int32)
    for ki in range(H):
        for kj in range(W_):
            idx = [i * W_ + j
                   for i in range(H) for j in range(W_)
                   if abs(i - ki) <= WIN_HALF and abs(j - kj) <= WIN_HALF]
            t[ki * W_ + kj, :len(idx)] = np.array(idx, dtype=np.int32)
    return t


_CIDX = _build_cidx()


def _sims_body(x_ref, lo_ref, hi_ref):
    for i in range(BPG):
        X = x_ref[i]
        n = jnp.sqrt(jnp.sum(X * X, axis=0))
        S = lax.dot_general(X, X, (((0,), (0,)), ((), ())),
                            preferred_element_type=jnp.float32,
                            precision=lax.Precision.HIGHEST)
        sim = S / jnp.maximum(n[:, None] * n[None, :], EPS)
        lo_ref[pl.ds(i * PR, P)] = sim[:, :128]
        hi_ref[pl.ds(i * PR, P)] = jnp.concatenate(
            [sim[:, 128:], jnp.full((P, 256 - P), NEG, jnp.float32)], axis=1)


def _merge_desc(ak, av, bk, bv):
    brk = lax.rev(bk, (0,))
    brv = lax.rev(bv, (0,))
    m = ak >= brk
    hk = jnp.where(m, ak, brk)
    hv = jnp.where(m, av, brv)
    return plsc.sort_key_val(hk, hv, descending=True)


def _sc_topk_body(lo_hbm, hi_hbm, cidx_hbm, out_hbm,
                  lo_v, hi_v, cidx_v, out_v):
    wid = lax.axis_index("s") * 2 + lax.axis_index("c")
    pltpu.sync_copy(lo_hbm.at[pl.ds(wid * PR, PR)], lo_v)
    pltpu.sync_copy(hi_hbm.at[pl.ds(wid * PR, PR)], hi_v)
    pltpu.sync_copy(cidx_hbm, cidx_v)
    lane = lax.broadcasted_iota(jnp.int32, (16,), 0)

    def one_pixel(p):
        psplat = lane * 0 + p
        parts = []
        for k in range(4):
            ci = cidx_v[pl.ds(p * NCAND + 16 * k, 16)]
            valid = ci >= 0
            safe = jnp.where(valid, ci, 0)
            in_lo = safe < 128
            g_lo = plsc.load_gather(lo_v, [psplat, jnp.where(in_lo, safe, 0)])
            g_hi = plsc.load_gather(hi_v, [psplat,
                                           jnp.where(in_lo, 0, safe - 128)])
            g = jnp.where(in_lo, g_lo, g_hi)
            key = jnp.where(valid, g, NEG)
            if k < 3:
                parts.append(plsc.sort_key_val(key, ci, descending=True))
            else:
                parts.append((key, ci))
        k01, v01 = _merge_desc(*parts[0], *parts[1])
        k23, v23 = _merge_desc(*parts[2], *parts[3])
        _, topv = _merge_desc(k01, v01, k23, v23)
        idx9 = jnp.where(lane < KK, topv, jnp.int32(2 ** 30))
        sidx, _ = plsc.sort_key_val(idx9, idx9)
        out_v[pl.ds(p * 128, 16)] = sidx

    def body(j, carry):
        one_pixel(2 * j)
        one_pixel(2 * j + 1)
        return carry

    lax.fori_loop(0, P // 2, body, 0)
    pltpu.sync_copy(out_v, out_hbm.at[pl.ds(wid * PR * 128, PR * 128)])


def _sc_topk(lo, hi, cidx):
    mesh = plsc.VectorSubcoreMesh(core_axis_name="c", subcore_axis_name="s")
    f = pl.kernel(
        _sc_topk_body,
        compiler_params=pltpu.CompilerParams(needs_layout_passes=False),
        out_type=jax.ShapeDtypeStruct((32 * PR * 128,), jnp.int32),
        mesh=mesh,
        scratch_types=[
            pltpu.VMEM((PR, 128), jnp.float32),
            pltpu.VMEM((PR, 128), jnp.float32),
            pltpu.VMEM((P * NCAND,), jnp.int32),
            pltpu.VMEM((PR * 128,), jnp.int32),
        ],
    )
    return f(lo, hi, cidx)


def _conv_body(x_ref, idx_ref, w_ref, o_ref):
    q_i = lax.broadcasted_iota(jnp.int32, (P, P), 1)
    for i in range(BPG):
        X = x_ref[i]
        acc = jnp.zeros((OUT, P), jnp.float32)
        for m in range(KK):
            im = idx_ref[i, :P, m][:, None]
            pm = (q_i == im).astype(jnp.float32)
            gm = lax.dot_general(pm, X, (((1,), (1,)), ((), ())),
                                 preferred_element_type=jnp.float32)
            acc = acc + lax.dot_general(
                w_ref[m], gm, (((0,), (1,)), ((), ())),
                preferred_element_type=jnp.float32)
        o_ref[i] = acc


def kernel(x, W):
    Bn = x.shape[0]
    x_flat = x.reshape(Bn, C, P)
    Wr = jnp.transpose(W.reshape(OUT, C, KK), (2, 1, 0))
    cidx = jnp.asarray(_CIDX.reshape(-1))

    lo, hi = pl.pallas_call(
        _sims_body,
        grid=(Bn // BPG,),
        in_specs=[pl.BlockSpec((BPG, C, P), lambda b: (b, 0, 0))],
        out_specs=[
            pl.BlockSpec((BPG * PR, 128), lambda b: (b, 0)),
            pl.BlockSpec((BPG * PR, 128), lambda b: (b, 0)),
        ],
        out_shape=[
            jax.ShapeDtypeStruct((Bn * PR, 128), jnp.float32),
            jax.ShapeDtypeStruct((Bn * PR, 128), jnp.float32),
        ],
    )(x_flat)

    idx = _sc_topk(lo, hi, cidx).reshape(Bn, PR, 128)

    out = pl.pallas_call(
        _conv_body,
        grid=(Bn // BPG,),
        in_specs=[
            pl.BlockSpec((BPG, C, P), lambda b: (b, 0, 0)),
            pl.BlockSpec((BPG, PR, 128), lambda b: (b, 0, 0)),
            pl.BlockSpec((KK, C, OUT), lambda b: (0, 0, 0)),
        ],
        out_specs=pl.BlockSpec((BPG, OUT, P), lambda b: (b, 0, 0)),
        out_shape=jax.ShapeDtypeStruct((Bn, OUT, P), jnp.float32),
    )(x_flat, idx, Wr)
    return out.reshape(Bn, OUT, H, W_)
